# Initial kernel scaffold; baseline (speedup 1.0000x reference)
import jax
import jax.numpy as jnp
from jax.experimental import pallas as pl


def kernel(soil_idx, crop_idx, weather_idx, E_soil, E_crop, E_weather, W_yield, b_yield, W_alloc, b_alloc):
    raise NotImplementedError("write your pallas kernel here")


# speedup vs baseline: 9.7603x; 9.7603x over previous
import jax
import jax.numpy as jnp
from jax import lax
from jax.experimental import pallas as pl
from jax.experimental.pallas import tpu as pltpu
from jax.experimental.pallas import tpu_sc as plsc

B, L = 4096, 50
N_TOK = B * L
D_SOIL, D_CROP, D_WEATHER = 128, 64, 32
D_FEAT = D_SOIL + D_CROP + D_WEATHER
VOCAB = 1000
VPAD = 1024
NCH = 8

NC, NS, LANES = 2, 16, 16
NW = NC * NS
CHUNK = N_TOK // NW
GROUPS = CHUNK // LANES


def _project_body(es_ref, ec_ref, ew_ref, w_ref, b_ref, out_ref):
    w = w_ref[:]
    out_ref[0:VPAD, :] = (
        jnp.dot(es_ref[:], w[0:D_SOIL, :], preferred_element_type=jnp.float32)
        + b_ref[:]
    )
    out_ref[VPAD:2 * VPAD, :] = jnp.dot(
        ec_ref[:], w[D_SOIL:D_SOIL + D_CROP, :],
        preferred_element_type=jnp.float32)
    out_ref[2 * VPAD:3 * VPAD, :] = jnp.dot(
        ew_ref[:], w[D_SOIL + D_CROP:D_FEAT, :],
        preferred_element_type=jnp.float32)


_project = pl.pallas_call(
    _project_body,
    out_shape=jax.ShapeDtypeStruct((3 * VPAD, NCH), jnp.float32),
)


def _sc_body(p_hbm, s_hbm, c_hbm, w_hbm, y_hbm, a_hbm,
             p_v, s_v, c_v, w_v, y_v, a_v):
    wid = lax.axis_index("s") * NC + lax.axis_index("c")
    base = wid * CHUNK
    pltpu.sync_copy(p_hbm, p_v)
    pltpu.sync_copy(s_hbm.at[pl.ds(base, CHUNK)], s_v)
    pltpu.sync_copy(c_hbm.at[pl.ds(base, CHUNK)], c_v)
    pltpu.sync_copy(w_hbm.at[pl.ds(base, CHUNK)], w_v)

    iota = lax.iota(jnp.int32, LANES)

    def body(g, carry):
        off = g * LANES
        s8 = s_v[pl.ds(off, LANES)] * NCH
        c8 = c_v[pl.ds(off, LANES)] * NCH + VPAD * NCH
        w8 = w_v[pl.ds(off, LANES)] * NCH + 2 * VPAD * NCH
        ch = []
        for c in range(7):
            ch.append(plsc.load_gather(p_v, [s8 + c])
                      + plsc.load_gather(p_v, [c8 + c])
                      + plsc.load_gather(p_v, [w8 + c]))
        y_v[pl.ds(off, LANES)] = ch[0]
        m = ch[1]
        for c in range(2, 7):
            m = jnp.maximum(m, ch[c])
        es = [jnp.exp(ch[c] - m) for c in range(1, 7)]
        tot = es[0]
        for e in es[1:]:
            tot = tot + e
        inv = 1.0 / tot
        t6 = (iota + off) * 6
        for j in range(6):
            plsc.store_scatter(a_v, [t6 + j], es[j] * inv)
        return carry

    lax.fori_loop(0, GROUPS, body, 0)

    pltpu.sync_copy(y_v, y_hbm.at[pl.ds(base, CHUNK)])
    pltpu.sync_copy(a_v, a_hbm.at[pl.ds(base * 6, CHUNK * 6)])


_sc_lookup = pl.kernel(
    _sc_body,
    out_type=[jax.ShapeDtypeStruct((N_TOK,), jnp.float32),
              jax.ShapeDtypeStruct((N_TOK * 6,), jnp.float32)],
    mesh=plsc.VectorSubcoreMesh(core_axis_name="c", subcore_axis_name="s"),
    compiler_params=pltpu.CompilerParams(needs_layout_passes=False),
    scratch_types=[
        pltpu.VMEM((3 * VPAD * NCH,), jnp.float32),
        pltpu.VMEM((CHUNK,), jnp.int32),
        pltpu.VMEM((CHUNK,), jnp.int32),
        pltpu.VMEM((CHUNK,), jnp.int32),
        pltpu.VMEM((CHUNK,), jnp.float32),
        pltpu.VMEM((CHUNK * 6,), jnp.float32),
    ],
)


def kernel(soil_idx, crop_idx, weather_idx, E_soil, E_crop, E_weather,
           W_yield, b_yield, W_alloc, b_alloc):
    f32 = jnp.float32
    es = jnp.pad(E_soil, ((0, VPAD - VOCAB), (0, 0)))
    ec = jnp.pad(E_crop, ((0, VPAD - VOCAB), (0, 0)))
    ew = jnp.pad(E_weather, ((0, VPAD - VOCAB), (0, 0)))
    wcat = jnp.concatenate(
        [W_yield, W_alloc, jnp.zeros((D_FEAT, 1), f32)], axis=1)
    bcat = jnp.concatenate(
        [b_yield, b_alloc, jnp.zeros((1,), f32)]).reshape(1, NCH)
    p = _project(es, ec, ew, wcat, bcat).reshape(-1)
    si = soil_idx.reshape(-1).astype(jnp.int32)
    ci = crop_idx.reshape(-1).astype(jnp.int32)
    wi = weather_idx.reshape(-1).astype(jnp.int32)
    y, a = _sc_lookup(p, si, ci, wi)
    return y.reshape(B, L, 1), a.reshape(B, L, 6)
